# Initial kernel scaffold; baseline (speedup 1.0000x reference)
import jax
import jax.numpy as jnp
from jax.experimental import pallas as pl


def kernel(logit_X, logit_E, Y, src, dst, W1, W2):
    raise NotImplementedError("write your pallas kernel here")


# speedup vs baseline: 34.2130x; 34.2130x over previous
import functools

import jax
import jax.numpy as jnp
from jax import lax
from jax.experimental import pallas as pl
from jax.experimental.pallas import tpu as pltpu
from jax.experimental.pallas import tpu_sc as plsc

N = 4096
C = 16
M = 65536
NE = 4
NY = 10
H = 128

NCORE = 2
NSUB = 16
NWORK = NCORE * NSUB
PAIRS = 2 * M
PER_TEC = PAIRS // NWORK
CHUNK = 128
N_CHUNKS = PER_TEC // CHUNK
ROWS_PER_TEC = N // NSUB


def _tc_sample_body(lx_ref, ux_ref, let_ref, uet_ref, src_ref, dst_ref,
                    w1_ref, h_ref, g_ref, s_ref):
    lx = lx_ref[...]
    gx = -jnp.log(-jnp.log(ux_ref[...]))
    mx = jnp.max(lx, axis=1, keepdims=True)
    ex = jnp.exp(lx - mx)
    px = ex / jnp.sum(ex, axis=1, keepdims=True)
    sx = jnp.log(px + 1e-20) + gx
    amax = jnp.max(sx, axis=1, keepdims=True)
    ii = lax.broadcasted_iota(jnp.int32, (N, C), 1)
    idxm = jnp.min(jnp.where(sx >= amax, ii, C), axis=1, keepdims=True)
    oh = (ii == idxm).astype(jnp.float32)
    h_ref[...] = jnp.dot(oh, w1_ref[...], preferred_element_type=jnp.float32)

    x0 = let_ref[0]
    x1 = let_ref[1]
    x2 = let_ref[2]
    x3 = let_ref[3]
    me = jnp.maximum(jnp.maximum(x0, x1), jnp.maximum(x2, x3))
    e0 = jnp.exp(x0 - me)
    e1 = jnp.exp(x1 - me)
    e2 = jnp.exp(x2 - me)
    e3 = jnp.exp(x3 - me)
    ssum = e0 + e1 + e2 + e3
    s0 = jnp.log(e0 / ssum + 1e-20) - jnp.log(-jnp.log(uet_ref[0]))
    s1 = jnp.log(e1 / ssum + 1e-20) - jnp.log(-jnp.log(uet_ref[1]))
    s2 = jnp.log(e2 / ssum + 1e-20) - jnp.log(-jnp.log(uet_ref[2]))
    s3 = jnp.log(e3 / ssum + 1e-20) - jnp.log(-jnp.log(uet_ref[3]))
    m = (s1 > s0) | (s2 > s0) | (s3 > s0)
    src = src_ref[...]
    dst = dst_ref[...]
    trash = jnp.full(src.shape, N, dtype=jnp.int32)
    g_ref[0] = dst
    g_ref[1] = src
    s_ref[0] = jnp.where(m, src, trash)
    s_ref[1] = jnp.where(m, dst, trash)


_tc_sample = pl.pallas_call(
    _tc_sample_body,
    out_shape=[
        jax.ShapeDtypeStruct((N, H), jnp.float32),
        jax.ShapeDtypeStruct((2, M // 128, 128), jnp.int32),
        jax.ShapeDtypeStruct((2, M // 128, 128), jnp.int32),
    ],
)


def _sc_agg_body(h_hbm, g_hbm, s_hbm, out_hbm, gidx_v, sidx_v, rows_v, agg_sh, sem):
    cid = lax.axis_index("c")
    sid = lax.axis_index("s")

    for hop in range(ROWS_PER_TEC // CHUNK):
        r0 = sid * ROWS_PER_TEC + hop * CHUNK
        pltpu.sync_copy(h_hbm.at[pl.ds(r0, CHUNK)], rows_v)
        pltpu.sync_copy(rows_v, agg_sh.at[pl.ds(r0, CHUNK)])
    plsc.subcore_barrier()

    wid = cid * NSUB + sid
    base = wid * PER_TEC

    def step(i, carry):
        off = base + i * CHUNK
        pltpu.sync_copy(g_hbm.at[pl.ds(off, CHUNK)], gidx_v)
        pltpu.async_copy(h_hbm.at[gidx_v], rows_v, sem).wait()
        pltpu.sync_copy(s_hbm.at[pl.ds(off, CHUNK)], sidx_v)
        pltpu.sync_copy(rows_v, agg_sh.at[sidx_v], add=True)
        return carry

    lax.fori_loop(0, N_CHUNKS, step, 0)
    plsc.subcore_barrier()

    for hop in range(ROWS_PER_TEC // CHUNK):
        r0 = sid * ROWS_PER_TEC + hop * CHUNK
        pltpu.sync_copy(agg_sh.at[pl.ds(r0, CHUNK)], rows_v)
        pltpu.sync_copy(rows_v, out_hbm.at[cid, pl.ds(r0, CHUNK)])


@functools.lru_cache(maxsize=None)
def _get_sc_agg():
    return pl.kernel(
        _sc_agg_body,
        out_type=jax.ShapeDtypeStruct((NCORE, N, H), jnp.float32),
        mesh=plsc.VectorSubcoreMesh(core_axis_name="c", subcore_axis_name="s",
                                    num_cores=NCORE, num_subcores=NSUB),
        scratch_types=[
            pltpu.VMEM((CHUNK,), jnp.int32),
            pltpu.VMEM((CHUNK,), jnp.int32),
            pltpu.VMEM((CHUNK, H), jnp.float32),
            pltpu.VMEM_SHARED((N + 8, H), jnp.float32),
            pltpu.SemaphoreType.DMA,
        ],
    )


def _tc_head_body(a_ref, h_ref, w2_ref, ohy_ref, loss_ref):
    z = jnp.maximum(a_ref[0] + a_ref[1] - h_ref[...], 0.0)
    ly = jnp.dot(z, w2_ref[...], preferred_element_type=jnp.float32)
    mx = jnp.max(ly, axis=1, keepdims=True)
    lse = jnp.log(jnp.sum(jnp.exp(ly - mx), axis=1, keepdims=True)) + mx
    logp = ly - lse
    loss_ref[...] = jnp.reshape(-jnp.sum(logp * ohy_ref[...]) / N, (1, 1))


_tc_head = pl.pallas_call(
    _tc_head_body,
    out_shape=jax.ShapeDtypeStruct((1, 1), jnp.float32),
)


def kernel(logit_X, logit_E, Y, src, dst, W1, W2):
    kX = jax.random.fold_in(jax.random.key(42), 0)
    kE = jax.random.fold_in(jax.random.key(42), 1)
    tiny = jnp.finfo(jnp.float32).tiny
    uX = jax.random.uniform(kX, (N, C), jnp.float32, minval=tiny, maxval=1.0)
    uE = jax.random.uniform(kE, (M, NE), jnp.float32, minval=tiny, maxval=1.0)
    leT = logit_E.T.reshape(NE, M // 128, 128)
    ueT = uE.T.reshape(NE, M // 128, 128)
    srcR = src.astype(jnp.int32).reshape(M // 128, 128)
    dstR = dst.astype(jnp.int32).reshape(M // 128, 128)
    h, g, s = _tc_sample(logit_X, uX, leT, ueT, srcR, dstR, W1)
    a = _get_sc_agg()(h, g.reshape(PAIRS), s.reshape(PAIRS))
    ohY = jax.nn.one_hot(Y, NY, dtype=jnp.float32)
    loss = _tc_head(a, h, W2, ohY)
    return loss.reshape(())


# speedup vs baseline: 52.6714x; 1.5395x over previous
import functools

import jax
import jax.numpy as jnp
from jax import lax
from jax.experimental import pallas as pl
from jax.experimental.pallas import tpu as pltpu
from jax.experimental.pallas import tpu_sc as plsc

N = 4096
C = 16
M = 65536
NE = 4
NY = 10
H = 128

NCORE = 2
NSUB = 16
NWORK = NCORE * NSUB
PAIRS = 2 * M
PER_TEC = PAIRS // NWORK
CHUNK = 128
N_CHUNKS = PER_TEC // CHUNK
ROWS_PER_TEC = N // NSUB


def _tc_sample_body(lx_ref, ux_ref, let_ref, uet_ref, src_ref, dst_ref,
                    w1_ref, h_ref, g_ref, s_ref):
    lx = lx_ref[...]
    gx = -jnp.log(-jnp.log(ux_ref[...]))
    mx = jnp.max(lx, axis=1, keepdims=True)
    ex = jnp.exp(lx - mx)
    px = ex / jnp.sum(ex, axis=1, keepdims=True)
    sx = jnp.log(px + 1e-20) + gx
    amax = jnp.max(sx, axis=1, keepdims=True)
    ii = lax.broadcasted_iota(jnp.int32, (N, C), 1)
    idxm = jnp.min(jnp.where(sx >= amax, ii, C), axis=1, keepdims=True)
    oh = (ii == idxm).astype(jnp.float32)
    h_ref[...] = jnp.dot(oh, w1_ref[...], preferred_element_type=jnp.float32)

    x0 = let_ref[0]
    x1 = let_ref[1]
    x2 = let_ref[2]
    x3 = let_ref[3]
    me = jnp.maximum(jnp.maximum(x0, x1), jnp.maximum(x2, x3))
    e0 = jnp.exp(x0 - me)
    e1 = jnp.exp(x1 - me)
    e2 = jnp.exp(x2 - me)
    e3 = jnp.exp(x3 - me)
    ssum = e0 + e1 + e2 + e3
    s0 = jnp.log(e0 / ssum + 1e-20) - jnp.log(-jnp.log(uet_ref[0]))
    s1 = jnp.log(e1 / ssum + 1e-20) - jnp.log(-jnp.log(uet_ref[1]))
    s2 = jnp.log(e2 / ssum + 1e-20) - jnp.log(-jnp.log(uet_ref[2]))
    s3 = jnp.log(e3 / ssum + 1e-20) - jnp.log(-jnp.log(uet_ref[3]))
    m = (s1 > s0) | (s2 > s0) | (s3 > s0)
    src = src_ref[...]
    dst = dst_ref[...]
    trash = jnp.full(src.shape, N, dtype=jnp.int32)
    g_ref[0] = dst
    g_ref[1] = src
    s_ref[0] = jnp.where(m, src, trash)
    s_ref[1] = jnp.where(m, dst, trash)


_tc_sample = pl.pallas_call(
    _tc_sample_body,
    out_shape=[
        jax.ShapeDtypeStruct((N, H), jnp.float32),
        jax.ShapeDtypeStruct((2, M // 128, 128), jnp.int32),
        jax.ShapeDtypeStruct((2, M // 128, 128), jnp.int32),
    ],
)


def _sc_agg_body(h_hbm, g_hbm, s_hbm, out_hbm, g_all, s_all, rows0, rows1,
                 agg_sh, sem0, sem1):
    cid = lax.axis_index("c")
    sid = lax.axis_index("s")
    wid = cid * NSUB + sid
    blk = wid * N_CHUNKS

    pltpu.sync_copy(g_hbm.at[pl.ds(blk, N_CHUNKS)], g_all)
    pltpu.sync_copy(s_hbm.at[pl.ds(blk, N_CHUNKS)], s_all)
    pltpu.async_copy(h_hbm.at[g_all.at[0]], rows0, sem0)

    for hop in range(ROWS_PER_TEC // CHUNK):
        r0 = sid * ROWS_PER_TEC + hop * CHUNK
        pltpu.sync_copy(h_hbm.at[pl.ds(r0, CHUNK)], rows1)
        pltpu.sync_copy(rows1, agg_sh.at[pl.ds(r0, CHUNK)])
    plsc.subcore_barrier()

    def step(i, carry):
        j0 = 2 * i
        j1 = 2 * i + 1
        d1 = pltpu.async_copy(h_hbm.at[g_all.at[j1]], rows1, sem1)
        pltpu.make_async_copy(h_hbm.at[pl.ds(0, CHUNK)], rows0, sem0).wait()
        pltpu.sync_copy(rows0, agg_sh.at[s_all.at[j0]], add=True)

        @pl.when(i < N_CHUNKS // 2 - 1)
        def _():
            pltpu.async_copy(h_hbm.at[g_all.at[j1 + 1]], rows0, sem0)

        d1.wait()
        pltpu.sync_copy(rows1, agg_sh.at[s_all.at[j1]], add=True)
        return carry

    lax.fori_loop(0, N_CHUNKS // 2, step, 0)
    plsc.subcore_barrier()

    for hop in range(ROWS_PER_TEC // CHUNK):
        r0 = sid * ROWS_PER_TEC + hop * CHUNK
        pltpu.sync_copy(agg_sh.at[pl.ds(r0, CHUNK)], rows0)
        pltpu.sync_copy(rows0, out_hbm.at[cid, pl.ds(r0, CHUNK)])


@functools.lru_cache(maxsize=None)
def _get_sc_agg():
    return pl.kernel(
        _sc_agg_body,
        out_type=jax.ShapeDtypeStruct((NCORE, N, H), jnp.float32),
        mesh=plsc.VectorSubcoreMesh(core_axis_name="c", subcore_axis_name="s",
                                    num_cores=NCORE, num_subcores=NSUB),
        scratch_types=[
            pltpu.VMEM((N_CHUNKS, CHUNK), jnp.int32),
            pltpu.VMEM((N_CHUNKS, CHUNK), jnp.int32),
            pltpu.VMEM((CHUNK, H), jnp.float32),
            pltpu.VMEM((CHUNK, H), jnp.float32),
            pltpu.VMEM_SHARED((N + 8, H), jnp.float32),
            pltpu.SemaphoreType.DMA,
            pltpu.SemaphoreType.DMA,
        ],
    )


def _tc_head_body(a_ref, h_ref, w2_ref, ohy_ref, loss_ref):
    z = jnp.maximum(a_ref[0] + a_ref[1] - h_ref[...], 0.0)
    ly = jnp.dot(z, w2_ref[...], preferred_element_type=jnp.float32)
    mx = jnp.max(ly, axis=1, keepdims=True)
    lse = jnp.log(jnp.sum(jnp.exp(ly - mx), axis=1, keepdims=True)) + mx
    logp = ly - lse
    loss_ref[...] = jnp.reshape(-jnp.sum(logp * ohy_ref[...]) / N, (1, 1))


_tc_head = pl.pallas_call(
    _tc_head_body,
    out_shape=jax.ShapeDtypeStruct((1, 1), jnp.float32),
)


def kernel(logit_X, logit_E, Y, src, dst, W1, W2):
    kX = jax.random.fold_in(jax.random.key(42), 0)
    kE = jax.random.fold_in(jax.random.key(42), 1)
    tiny = jnp.finfo(jnp.float32).tiny
    uX = jax.random.uniform(kX, (N, C), jnp.float32, minval=tiny, maxval=1.0)
    uE = jax.random.uniform(kE, (M, NE), jnp.float32, minval=tiny, maxval=1.0)
    leT = logit_E.T.reshape(NE, M // 128, 128)
    ueT = uE.T.reshape(NE, M // 128, 128)
    srcR = src.astype(jnp.int32).reshape(M // 128, 128)
    dstR = dst.astype(jnp.int32).reshape(M // 128, 128)
    h, g, s = _tc_sample(logit_X, uX, leT, ueT, srcR, dstR, W1)
    a = _get_sc_agg()(h, g.reshape(PAIRS // CHUNK, CHUNK),
                      s.reshape(PAIRS // CHUNK, CHUNK))
    ohY = jax.nn.one_hot(Y, NY, dtype=jnp.float32)
    loss = _tc_head(a, h, W2, ohY)
    return loss.reshape(())


# speedup vs baseline: 53.4488x; 1.0148x over previous
import functools

import jax
import jax.numpy as jnp
from jax import lax
from jax.experimental import pallas as pl
from jax.experimental.pallas import tpu as pltpu
from jax.experimental.pallas import tpu_sc as plsc

N = 4096
C = 16
M = 65536
NE = 4
NY = 10
H = 128

NCORE = 2
NSUB = 16
NWORK = NCORE * NSUB
PAIRS = 2 * M
PER_TEC = PAIRS // NWORK
CHUNK = 128
N_CHUNKS = PER_TEC // CHUNK
ROWS_PER_TEC = N // NSUB


def _tc_sample_body(lx_ref, ux_ref, let_ref, uet_ref, src_ref, dst_ref,
                    w1_ref, h_ref, g_ref, s_ref):
    lx = lx_ref[...]
    gx = -jnp.log(-jnp.log(ux_ref[...]))
    sx = lx + gx
    amax = jnp.max(sx, axis=1, keepdims=True)
    ii = lax.broadcasted_iota(jnp.int32, (N, C), 1)
    idxm = jnp.min(jnp.where(sx >= amax, ii, C), axis=1, keepdims=True)
    oh = (ii == idxm).astype(jnp.float32)
    h_ref[...] = jnp.dot(oh, w1_ref[...], preferred_element_type=jnp.float32)

    s0 = let_ref[0] - jnp.log(-jnp.log(uet_ref[0]))
    s1 = let_ref[1] - jnp.log(-jnp.log(uet_ref[1]))
    s2 = let_ref[2] - jnp.log(-jnp.log(uet_ref[2]))
    s3 = let_ref[3] - jnp.log(-jnp.log(uet_ref[3]))
    m = (s1 > s0) | (s2 > s0) | (s3 > s0)
    src = src_ref[...]
    dst = dst_ref[...]
    trash = jnp.full(src.shape, N, dtype=jnp.int32)
    g_ref[0] = dst
    g_ref[1] = src
    s_ref[0] = jnp.where(m, src, trash)
    s_ref[1] = jnp.where(m, dst, trash)


_tc_sample = pl.pallas_call(
    _tc_sample_body,
    out_shape=[
        jax.ShapeDtypeStruct((N, H), jnp.float32),
        jax.ShapeDtypeStruct((2, M // 128, 128), jnp.int32),
        jax.ShapeDtypeStruct((2, M // 128, 128), jnp.int32),
    ],
)


def _sc_agg_body(h_hbm, g_hbm, s_hbm, out_hbm, g_all, s_all, rows0, rows1,
                 agg_sh, sem0, sem1):
    cid = lax.axis_index("c")
    sid = lax.axis_index("s")
    wid = cid * NSUB + sid
    blk = wid * N_CHUNKS

    pltpu.sync_copy(g_hbm.at[pl.ds(blk, N_CHUNKS)], g_all)
    pltpu.sync_copy(s_hbm.at[pl.ds(blk, N_CHUNKS)], s_all)
    pltpu.async_copy(h_hbm.at[g_all.at[0]], rows0, sem0)

    for hop in range(ROWS_PER_TEC // CHUNK):
        r0 = sid * ROWS_PER_TEC + hop * CHUNK
        pltpu.sync_copy(h_hbm.at[pl.ds(r0, CHUNK)], rows1)
        pltpu.sync_copy(rows1, agg_sh.at[pl.ds(r0, CHUNK)])
    plsc.subcore_barrier()

    def step(i, carry):
        j0 = 2 * i
        j1 = 2 * i + 1
        d1 = pltpu.async_copy(h_hbm.at[g_all.at[j1]], rows1, sem1)
        pltpu.make_async_copy(h_hbm.at[pl.ds(0, CHUNK)], rows0, sem0).wait()
        pltpu.sync_copy(rows0, agg_sh.at[s_all.at[j0]], add=True)

        @pl.when(i < N_CHUNKS // 2 - 1)
        def _():
            pltpu.async_copy(h_hbm.at[g_all.at[j1 + 1]], rows0, sem0)

        d1.wait()
        pltpu.sync_copy(rows1, agg_sh.at[s_all.at[j1]], add=True)
        return carry

    lax.fori_loop(0, N_CHUNKS // 2, step, 0)
    plsc.subcore_barrier()

    for hop in range(ROWS_PER_TEC // CHUNK):
        r0 = sid * ROWS_PER_TEC + hop * CHUNK
        pltpu.sync_copy(agg_sh.at[pl.ds(r0, CHUNK)], rows0)
        pltpu.sync_copy(rows0, out_hbm.at[cid, pl.ds(r0, CHUNK)])


@functools.lru_cache(maxsize=None)
def _get_sc_agg():
    return pl.kernel(
        _sc_agg_body,
        out_type=jax.ShapeDtypeStruct((NCORE, N, H), jnp.float32),
        mesh=plsc.VectorSubcoreMesh(core_axis_name="c", subcore_axis_name="s",
                                    num_cores=NCORE, num_subcores=NSUB),
        scratch_types=[
            pltpu.VMEM((N_CHUNKS, CHUNK), jnp.int32),
            pltpu.VMEM((N_CHUNKS, CHUNK), jnp.int32),
            pltpu.VMEM((CHUNK, H), jnp.float32),
            pltpu.VMEM((CHUNK, H), jnp.float32),
            pltpu.VMEM_SHARED((N + 8, H), jnp.float32),
            pltpu.SemaphoreType.DMA,
            pltpu.SemaphoreType.DMA,
        ],
    )


def _tc_head_body(a_ref, h_ref, w2_ref, ohy_ref, loss_ref):
    z = jnp.maximum(a_ref[0] + a_ref[1] - h_ref[...], 0.0)
    ly = jnp.dot(z, w2_ref[...], preferred_element_type=jnp.float32)
    mx = jnp.max(ly, axis=1, keepdims=True)
    lse = jnp.log(jnp.sum(jnp.exp(ly - mx), axis=1, keepdims=True)) + mx
    logp = ly - lse
    loss_ref[...] = jnp.reshape(-jnp.sum(logp * ohy_ref[...]) / N, (1, 1))


_tc_head = pl.pallas_call(
    _tc_head_body,
    out_shape=jax.ShapeDtypeStruct((1, 1), jnp.float32),
)


def kernel(logit_X, logit_E, Y, src, dst, W1, W2):
    kX = jax.random.fold_in(jax.random.key(42), 0)
    kE = jax.random.fold_in(jax.random.key(42), 1)
    tiny = jnp.finfo(jnp.float32).tiny
    uX = jax.random.uniform(kX, (N, C), jnp.float32, minval=tiny, maxval=1.0)
    uE = jax.random.uniform(kE, (M, NE), jnp.float32, minval=tiny, maxval=1.0)
    leT = logit_E.T.reshape(NE, M // 128, 128)
    ueT = uE.T.reshape(NE, M // 128, 128)
    srcR = src.astype(jnp.int32).reshape(M // 128, 128)
    dstR = dst.astype(jnp.int32).reshape(M // 128, 128)
    h, g, s = _tc_sample(logit_X, uX, leT, ueT, srcR, dstR, W1)
    a = _get_sc_agg()(h, g.reshape(PAIRS // CHUNK, CHUNK),
                      s.reshape(PAIRS // CHUNK, CHUNK))
    ohY = jax.nn.one_hot(Y, NY, dtype=jnp.float32)
    loss = _tc_head(a, h, W2, ohY)
    return loss.reshape(())


# speedup vs baseline: 53.6945x; 1.0046x over previous
import functools

import jax
import jax.numpy as jnp
from jax import lax
from jax.experimental import pallas as pl
from jax.experimental.pallas import tpu as pltpu
from jax.experimental.pallas import tpu_sc as plsc

N = 4096
C = 16
M = 65536
NE = 4
NY = 10
H = 128

NCORE = 2
NSUB = 16
NWORK = NCORE * NSUB
PAIRS = 2 * M
PER_TEC = PAIRS // NWORK
CHUNK = 128
N_CHUNKS = PER_TEC // CHUNK
ROWS_PER_TEC = N // NSUB


def _tc_sample_body(lx_ref, ux_ref, let_ref, uet_ref, src_ref, dst_ref,
                    w1_ref, h_ref, g_ref, s_ref):
    lx = lx_ref[...]
    gx = -jnp.log(-jnp.log(ux_ref[...]))
    sx = lx + gx
    amax = jnp.max(sx, axis=1, keepdims=True)
    ii = lax.broadcasted_iota(jnp.int32, (N, C), 1)
    idxm = jnp.min(jnp.where(sx >= amax, ii, C), axis=1, keepdims=True)
    oh = (ii == idxm).astype(jnp.float32)
    h_ref[...] = jnp.dot(oh, w1_ref[...], preferred_element_type=jnp.float32)

    s0 = let_ref[0] - jnp.log(-jnp.log(uet_ref[0]))
    s1 = let_ref[1] - jnp.log(-jnp.log(uet_ref[1]))
    s2 = let_ref[2] - jnp.log(-jnp.log(uet_ref[2]))
    s3 = let_ref[3] - jnp.log(-jnp.log(uet_ref[3]))
    m = (s1 > s0) | (s2 > s0) | (s3 > s0)
    src = src_ref[...]
    dst = dst_ref[...]
    trash = jnp.full(src.shape, N, dtype=jnp.int32)
    g_ref[0] = dst
    g_ref[1] = src
    s_ref[0] = jnp.where(m, src, trash)
    s_ref[1] = jnp.where(m, dst, trash)


_tc_sample = pl.pallas_call(
    _tc_sample_body,
    out_shape=[
        jax.ShapeDtypeStruct((N, H), jnp.float32),
        jax.ShapeDtypeStruct((2, M // 128, 128), jnp.int32),
        jax.ShapeDtypeStruct((2, M // 128, 128), jnp.int32),
    ],
)


_NBUF = 4


def _sc_agg_body(h_hbm, g_hbm, s_hbm, out_hbm, g_all, s_all, rows, gsems, ssems,
                 agg_sh):
    cid = lax.axis_index("c")
    sid = lax.axis_index("s")
    wid = cid * NSUB + sid
    blk = wid * N_CHUNKS

    pltpu.sync_copy(g_hbm.at[pl.ds(blk, N_CHUNKS)], g_all)
    pltpu.sync_copy(s_hbm.at[pl.ds(blk, N_CHUNKS)], s_all)
    for k in range(_NBUF):
        pltpu.async_copy(h_hbm.at[g_all.at[k]], rows[k], gsems[k])

    for hop in range(ROWS_PER_TEC // CHUNK):
        r0 = sid * ROWS_PER_TEC + hop * CHUNK
        pltpu.sync_copy(h_hbm.at[pl.ds(r0, CHUNK)], agg_sh.at[pl.ds(r0, CHUNK)])
    plsc.subcore_barrier()

    def step(i, carry):
        scat = []
        for k in range(_NBUF):
            j = _NBUF * i + k
            pltpu.make_async_copy(
                h_hbm.at[pl.ds(0, CHUNK)], rows[k], gsems[k]).wait()
            scat.append(pltpu.async_copy(
                rows[k], agg_sh.at[s_all.at[j]], ssems[k], add=True))
        for k in range(_NBUF):
            scat[k].wait()

            @pl.when(i < N_CHUNKS // _NBUF - 1)
            def _():
                pltpu.async_copy(
                    h_hbm.at[g_all.at[_NBUF * (i + 1) + k]], rows[k], gsems[k])
        return carry

    lax.fori_loop(0, N_CHUNKS // _NBUF, step, 0)
    plsc.subcore_barrier()

    for hop in range(ROWS_PER_TEC // CHUNK):
        r0 = sid * ROWS_PER_TEC + hop * CHUNK
        pltpu.sync_copy(agg_sh.at[pl.ds(r0, CHUNK)], rows[0])
        pltpu.sync_copy(rows[0], out_hbm.at[cid, pl.ds(r0, CHUNK)])


@functools.lru_cache(maxsize=None)
def _get_sc_agg():
    return pl.kernel(
        _sc_agg_body,
        out_type=jax.ShapeDtypeStruct((NCORE, N, H), jnp.float32),
        mesh=plsc.VectorSubcoreMesh(core_axis_name="c", subcore_axis_name="s",
                                    num_cores=NCORE, num_subcores=NSUB),
        scratch_types=[
            pltpu.VMEM((N_CHUNKS, CHUNK), jnp.int32),
            pltpu.VMEM((N_CHUNKS, CHUNK), jnp.int32),
            [pltpu.VMEM((CHUNK, H), jnp.float32) for _ in range(_NBUF)],
            [pltpu.SemaphoreType.DMA for _ in range(_NBUF)],
            [pltpu.SemaphoreType.DMA for _ in range(_NBUF)],
            pltpu.VMEM_SHARED((N + 8, H), jnp.float32),
        ],
    )


def _tc_head_body(a_ref, h_ref, w2_ref, ohy_ref, loss_ref):
    z = jnp.maximum(a_ref[0] + a_ref[1] - h_ref[...], 0.0)
    ly = jnp.dot(z, w2_ref[...], preferred_element_type=jnp.float32)
    mx = jnp.max(ly, axis=1, keepdims=True)
    lse = jnp.log(jnp.sum(jnp.exp(ly - mx), axis=1, keepdims=True)) + mx
    logp = ly - lse
    loss_ref[...] = jnp.reshape(-jnp.sum(logp * ohy_ref[...]) / N, (1, 1))


_tc_head = pl.pallas_call(
    _tc_head_body,
    out_shape=jax.ShapeDtypeStruct((1, 1), jnp.float32),
)


def kernel(logit_X, logit_E, Y, src, dst, W1, W2):
    kX = jax.random.fold_in(jax.random.key(42), 0)
    kE = jax.random.fold_in(jax.random.key(42), 1)
    tiny = jnp.finfo(jnp.float32).tiny
    uX = jax.random.uniform(kX, (N, C), jnp.float32, minval=tiny, maxval=1.0)
    uE = jax.random.uniform(kE, (M, NE), jnp.float32, minval=tiny, maxval=1.0)
    leT = logit_E.T.reshape(NE, M // 128, 128)
    ueT = uE.T.reshape(NE, M // 128, 128)
    srcR = src.astype(jnp.int32).reshape(M // 128, 128)
    dstR = dst.astype(jnp.int32).reshape(M // 128, 128)
    h, g, s = _tc_sample(logit_X, uX, leT, ueT, srcR, dstR, W1)
    a = _get_sc_agg()(h, g.reshape(PAIRS // CHUNK, CHUNK),
                      s.reshape(PAIRS // CHUNK, CHUNK))
    ohY = jax.nn.one_hot(Y, NY, dtype=jnp.float32)
    loss = _tc_head(a, h, W2, ohY)
    return loss.reshape(())


# speedup vs baseline: 55.6570x; 1.0366x over previous
import functools

import jax
import jax.numpy as jnp
from jax import lax
from jax.experimental import pallas as pl
from jax.experimental.pallas import tpu as pltpu
from jax.experimental.pallas import tpu_sc as plsc

N = 4096
C = 16
M = 65536
NE = 4
NY = 10
H = 128

NCORE = 2
NSUB = 16
NWORK = NCORE * NSUB
PAIRS = 2 * M
PER_TEC = PAIRS // NWORK
CHUNK = 128
N_CHUNKS = PER_TEC // CHUNK
ROWS_PER_TEC = N // NSUB


_KX = (0x6D3E048F, 0x1022172D)
_KE = (0x03D7B32D, 0xADD083F4)


def _tf_uniform(key, flat_idx):
    k0 = jnp.uint32(key[0])
    k1 = jnp.uint32(key[1])
    ks = (k0, k1, k0 ^ k1 ^ jnp.uint32(0x1BD11BDA))
    rot = ((13, 15, 26, 6), (17, 29, 16, 24))
    x1 = flat_idx.astype(jnp.uint32)
    x0 = jnp.zeros_like(x1) + ks[0]
    x1 = x1 + ks[1]
    for i in range(5):
        for r in rot[i % 2]:
            x0 = x0 + x1
            x1 = (x1 << jnp.uint32(r)) | (x1 >> jnp.uint32(32 - r))
            x1 = x1 ^ x0
        x0 = x0 + ks[(i + 1) % 3]
        x1 = x1 + ks[(i + 2) % 3] + jnp.uint32(i + 1)
    bits = x0 ^ x1
    fl = lax.bitcast_convert_type(
        (bits >> jnp.uint32(9)) | jnp.uint32(0x3F800000), jnp.float32) - 1.0
    tiny = jnp.finfo(jnp.float32).tiny
    return jnp.maximum(tiny, fl * (1.0 - tiny) + tiny)


def _tc_sample_body(lx_ref, let_ref, src_ref, dst_ref, w1_ref,
                    h_ref, g_ref, s_ref):
    lx = lx_ref[...]
    ii = lax.broadcasted_iota(jnp.int32, (N, C), 1)
    ir = lax.broadcasted_iota(jnp.int32, (N, C), 0)
    ux = _tf_uniform(_KX, ir * C + ii)
    gx = -jnp.log(-jnp.log(ux))
    sx = lx + gx
    amax = jnp.max(sx, axis=1, keepdims=True)
    idxm = jnp.min(jnp.where(sx >= amax, ii, C), axis=1, keepdims=True)
    oh = (ii == idxm).astype(jnp.float32)
    h_ref[...] = jnp.dot(oh, w1_ref[...], preferred_element_type=jnp.float32)

    er = lax.broadcasted_iota(jnp.int32, (M // 128, 128), 0)
    el = lax.broadcasted_iota(jnp.int32, (M // 128, 128), 1)
    jj4 = (er * 128 + el) * NE
    sc = []
    for c in range(NE):
        uc = _tf_uniform(_KE, jj4 + c)
        sc.append(let_ref[c] - jnp.log(-jnp.log(uc)))
    s0, s1, s2, s3 = sc
    m = (s1 > s0) | (s2 > s0) | (s3 > s0)
    src = src_ref[...]
    dst = dst_ref[...]
    trash = jnp.full(src.shape, N, dtype=jnp.int32)
    g_ref[0] = dst
    g_ref[1] = src
    s_ref[0] = jnp.where(m, src, trash)
    s_ref[1] = jnp.where(m, dst, trash)


_tc_sample = pl.pallas_call(
    _tc_sample_body,
    out_shape=[
        jax.ShapeDtypeStruct((N, H), jnp.float32),
        jax.ShapeDtypeStruct((2, M // 128, 128), jnp.int32),
        jax.ShapeDtypeStruct((2, M // 128, 128), jnp.int32),
    ],
)


_NBUF = 4


def _sc_agg_body(h_hbm, g_hbm, s_hbm, out_hbm, g_all, s_all, rows, gsems, ssems,
                 agg_sh):
    cid = lax.axis_index("c")
    sid = lax.axis_index("s")
    wid = cid * NSUB + sid
    blk = wid * N_CHUNKS

    pltpu.sync_copy(g_hbm.at[pl.ds(blk, N_CHUNKS)], g_all)
    pltpu.sync_copy(s_hbm.at[pl.ds(blk, N_CHUNKS)], s_all)
    for k in range(_NBUF):
        pltpu.async_copy(h_hbm.at[g_all.at[k]], rows[k], gsems[k])

    for hop in range(ROWS_PER_TEC // CHUNK):
        r0 = sid * ROWS_PER_TEC + hop * CHUNK
        pltpu.sync_copy(h_hbm.at[pl.ds(r0, CHUNK)], agg_sh.at[pl.ds(r0, CHUNK)])
    plsc.subcore_barrier()

    def step(i, carry):
        scat = []
        for k in range(_NBUF):
            j = _NBUF * i + k
            pltpu.make_async_copy(
                h_hbm.at[pl.ds(0, CHUNK)], rows[k], gsems[k]).wait()
            scat.append(pltpu.async_copy(
                rows[k], agg_sh.at[s_all.at[j]], ssems[k], add=True))
        for k in range(_NBUF):
            scat[k].wait()

            @pl.when(i < N_CHUNKS // _NBUF - 1)
            def _():
                pltpu.async_copy(
                    h_hbm.at[g_all.at[_NBUF * (i + 1) + k]], rows[k], gsems[k])
        return carry

    lax.fori_loop(0, N_CHUNKS // _NBUF, step, 0)
    plsc.subcore_barrier()

    for hop in range(ROWS_PER_TEC // CHUNK):
        r0 = sid * ROWS_PER_TEC + hop * CHUNK
        pltpu.sync_copy(agg_sh.at[pl.ds(r0, CHUNK)], rows[0])
        pltpu.sync_copy(rows[0], out_hbm.at[cid, pl.ds(r0, CHUNK)])


@functools.lru_cache(maxsize=None)
def _get_sc_agg():
    return pl.kernel(
        _sc_agg_body,
        out_type=jax.ShapeDtypeStruct((NCORE, N, H), jnp.float32),
        mesh=plsc.VectorSubcoreMesh(core_axis_name="c", subcore_axis_name="s",
                                    num_cores=NCORE, num_subcores=NSUB),
        scratch_types=[
            pltpu.VMEM((N_CHUNKS, CHUNK), jnp.int32),
            pltpu.VMEM((N_CHUNKS, CHUNK), jnp.int32),
            [pltpu.VMEM((CHUNK, H), jnp.float32) for _ in range(_NBUF)],
            [pltpu.SemaphoreType.DMA for _ in range(_NBUF)],
            [pltpu.SemaphoreType.DMA for _ in range(_NBUF)],
            pltpu.VMEM_SHARED((N + 8, H), jnp.float32),
        ],
    )


def _tc_head_body(a_ref, h_ref, w2_ref, ohy_ref, loss_ref):
    z = jnp.maximum(a_ref[0] + a_ref[1] - h_ref[...], 0.0)
    ly = jnp.dot(z, w2_ref[...], preferred_element_type=jnp.float32)
    mx = jnp.max(ly, axis=1, keepdims=True)
    lse = jnp.log(jnp.sum(jnp.exp(ly - mx), axis=1, keepdims=True)) + mx
    logp = ly - lse
    loss_ref[...] = jnp.reshape(-jnp.sum(logp * ohy_ref[...]) / N, (1, 1))


_tc_head = pl.pallas_call(
    _tc_head_body,
    out_shape=jax.ShapeDtypeStruct((1, 1), jnp.float32),
)


def kernel(logit_X, logit_E, Y, src, dst, W1, W2):
    leT = logit_E.T.reshape(NE, M // 128, 128)
    srcR = src.astype(jnp.int32).reshape(M // 128, 128)
    dstR = dst.astype(jnp.int32).reshape(M // 128, 128)
    h, g, s = _tc_sample(logit_X, leT, srcR, dstR, W1)
    a = _get_sc_agg()(h, g.reshape(PAIRS // CHUNK, CHUNK),
                      s.reshape(PAIRS // CHUNK, CHUNK))
    ohY = jax.nn.one_hot(Y, NY, dtype=jnp.float32)
    loss = _tc_head(a, h, W2, ohY)
    return loss.reshape(())
